# baseline (speedup 1.0000x reference)
import functools

import jax
import jax.numpy as jnp
from jax.experimental import pallas as pl
from jax.experimental.pallas import tpu as pltpu

D_MODEL = 1024
D_FF = 4096
E = 8
TOPK = 2
N = 2048

T = 128
NTOT = N * TOPK
NUM_TILES = NTOT // T + E
NUM_ROWS = NUM_TILES * T
F_BLK = 512
NUM_F = D_FF // F_BLK


def _gate_body(x_ref, wg_ref, w_ref, e_ref):
    logits = jax.lax.dot_general(
        x_ref[...], wg_ref[...], (((1,), (1,)), ((), ())),
        preferred_element_type=jnp.float32)
    ids = jax.lax.broadcasted_iota(jnp.int32, logits.shape, 1)
    v1 = jnp.max(logits, axis=1, keepdims=True)
    i1 = jnp.min(jnp.where(logits == v1, ids, E), axis=1, keepdims=True)
    l2 = jnp.where(ids == i1, -jnp.inf, logits)
    v2 = jnp.max(l2, axis=1, keepdims=True)
    i2 = jnp.min(jnp.where(l2 == v2, ids, E), axis=1, keepdims=True)
    s = jnp.exp(v2 - v1)
    w_ref[...] = jnp.concatenate([1.0 / (1.0 + s), s / (1.0 + s)], axis=1)
    e_ref[...] = jnp.concatenate([i1, i2], axis=1)


def _gate(x, Wg, interpret=False):
    return pl.pallas_call(
        _gate_body,
        out_shape=(jax.ShapeDtypeStruct((N, TOPK), jnp.float32),
                   jax.ShapeDtypeStruct((N, TOPK), jnp.int32)),
        interpret=interpret,
    )(x, Wg)


def _dispatch_body(tok_ref, x_ref, xs_ref):
    t = pl.program_id(0)

    def body(r, _):
        tk = tok_ref[t * T + r]
        xs_ref[pl.ds(r, 1), :] = x_ref[pl.ds(tk, 1), :]
        return 0

    jax.lax.fori_loop(0, T, body, 0)


def _dispatch(x, row_token, interpret=False):
    grid_spec = pltpu.PrefetchScalarGridSpec(
        num_scalar_prefetch=1,
        grid=(NUM_TILES,),
        in_specs=[pl.BlockSpec((N, D_MODEL), lambda t, tok: (0, 0))],
        out_specs=pl.BlockSpec((T, D_MODEL), lambda t, tok: (t, 0)),
    )
    return pl.pallas_call(
        _dispatch_body,
        grid_spec=grid_spec,
        out_shape=jax.ShapeDtypeStruct((NUM_ROWS, D_MODEL), jnp.float32),
        interpret=interpret,
    )(row_token, x)


def _moe_body(te_ref, tv_ref, xs_ref, w_ref, v_ref, w2_ref, rw_ref, out_ref):
    t = pl.program_id(0)
    f = pl.program_id(1)

    @pl.when(f == 0)
    def _zero():
        out_ref[...] = jnp.zeros_like(out_ref)

    @pl.when(tv_ref[t] == 1)
    def _compute():
        a = xs_ref[...]
        h1 = jax.lax.dot_general(
            a, w_ref[0], (((1,), (1,)), ((), ())),
            preferred_element_type=jnp.float32)
        h2 = jax.lax.dot_general(
            a, v_ref[0], (((1,), (1,)), ((), ())),
            preferred_element_type=jnp.float32)
        h = (h1 * jax.nn.sigmoid(h1)) * h2
        out_ref[...] += jax.lax.dot_general(
            h, w2_ref[0], (((1,), (1,)), ((), ())),
            preferred_element_type=jnp.float32)

    @pl.when(f == NUM_F - 1)
    def _scale():
        out_ref[...] *= rw_ref[0, 0, :][:, None]


def _moe_gemm(xs, W, V, W2, row_weight, tile_expert, tile_valid,
              interpret=False):
    grid_spec = pltpu.PrefetchScalarGridSpec(
        num_scalar_prefetch=2,
        grid=(NUM_TILES, NUM_F),
        in_specs=[
            pl.BlockSpec((T, D_MODEL), lambda t, f, te, tv: (t, 0)),
            pl.BlockSpec((1, F_BLK, D_MODEL), lambda t, f, te, tv: (te[t], f, 0)),
            pl.BlockSpec((1, F_BLK, D_MODEL), lambda t, f, te, tv: (te[t], f, 0)),
            pl.BlockSpec((1, D_MODEL, F_BLK), lambda t, f, te, tv: (te[t], 0, f)),
            pl.BlockSpec((1, 1, T), lambda t, f, te, tv: (t, 0, 0)),
        ],
        out_specs=pl.BlockSpec((T, D_MODEL), lambda t, f, te, tv: (t, 0)),
    )
    return pl.pallas_call(
        _moe_body,
        grid_spec=grid_spec,
        out_shape=jax.ShapeDtypeStruct((NUM_ROWS, D_MODEL), jnp.float32),
        interpret=interpret,
    )(tile_expert, tile_valid, xs, W, V, W2,
      row_weight.reshape(NUM_TILES, 1, T))


def _combine_body(tok_ref, outs_ref, y_ref):
    t = pl.program_id(0)

    @pl.when(t == 0)
    def _zero():
        y_ref[...] = jnp.zeros_like(y_ref)

    def body(r, _):
        tk = tok_ref[t * T + r]
        y_ref[pl.ds(tk, 1), :] += outs_ref[pl.ds(r, 1), :]
        return 0

    jax.lax.fori_loop(0, T, body, 0)


def _combine(outs, row_token, interpret=False):
    grid_spec = pltpu.PrefetchScalarGridSpec(
        num_scalar_prefetch=1,
        grid=(NUM_TILES,),
        in_specs=[pl.BlockSpec((T, D_MODEL), lambda t, tok: (t, 0))],
        out_specs=pl.BlockSpec((N, D_MODEL), lambda t, tok: (0, 0)),
    )
    return pl.pallas_call(
        _combine_body,
        grid_spec=grid_spec,
        out_shape=jax.ShapeDtypeStruct((N, D_MODEL), jnp.float32),
        interpret=interpret,
    )(row_token, outs)


def _routing_metadata(weights, experts):
    e_flat = experts.reshape(-1).astype(jnp.int32)
    w_flat = weights.reshape(-1)
    tok = jnp.arange(NTOT, dtype=jnp.int32) // TOPK
    order = jnp.argsort(e_flat, stable=True)
    e_sorted = e_flat[order]
    counts = jnp.bincount(e_flat, length=E).astype(jnp.int32)
    tiles_per_e = (counts + T - 1) // T
    tile_start = jnp.concatenate(
        [jnp.zeros(1, jnp.int32), jnp.cumsum(tiles_per_e)])
    count_start = jnp.concatenate(
        [jnp.zeros(1, jnp.int32), jnp.cumsum(counts)])
    k = jnp.arange(NTOT, dtype=jnp.int32)
    row = tile_start[e_sorted] * T + (k - count_start[e_sorted])
    row_token = jnp.zeros(NUM_ROWS, jnp.int32).at[row].set(tok[order])
    row_weight = jnp.zeros(NUM_ROWS, jnp.float32).at[row].set(w_flat[order])
    total_tiles = tile_start[E]
    g = jnp.minimum(jnp.arange(NUM_TILES, dtype=jnp.int32), total_tiles - 1)
    tile_expert = (jnp.searchsorted(tile_start, g, side="right") - 1
                   ).astype(jnp.int32)
    tile_valid = (jnp.arange(NUM_TILES, dtype=jnp.int32)
                  < total_tiles).astype(jnp.int32)
    return row_token, row_weight, tile_expert, tile_valid


@functools.partial(jax.jit, static_argnames=("interpret",))
def kernel(x, Wg, W, V, W2, interpret=False):
    weights, experts = _gate(x, Wg, interpret)
    row_token, row_weight, tile_expert, tile_valid = _routing_metadata(
        weights, experts)
    xs = _dispatch(x, row_token, interpret)
    outs = _moe_gemm(xs, W, V, W2, row_weight, tile_expert, tile_valid,
                     interpret)
    return _combine(outs, row_token, interpret)


# speedup vs baseline: 1.1522x; 1.1522x over previous
import functools

import jax
import jax.numpy as jnp
from jax.experimental import pallas as pl
from jax.experimental.pallas import tpu as pltpu

D_MODEL = 1024
D_FF = 4096
E = 8
TOPK = 2
N = 2048

T = 128
NTOT = N * TOPK
NUM_TILES = NTOT // T + E
NUM_ROWS = NUM_TILES * T
F_BLK = 512
NUM_F = D_FF // F_BLK


def _gate_body(x_ref, wg_ref, w_ref, e_ref):
    logits = jax.lax.dot_general(
        x_ref[...], wg_ref[...], (((1,), (1,)), ((), ())),
        preferred_element_type=jnp.float32)
    ids = jax.lax.broadcasted_iota(jnp.int32, logits.shape, 1)
    v1 = jnp.max(logits, axis=1, keepdims=True)
    i1 = jnp.min(jnp.where(logits == v1, ids, E), axis=1, keepdims=True)
    l2 = jnp.where(ids == i1, -jnp.inf, logits)
    v2 = jnp.max(l2, axis=1, keepdims=True)
    i2 = jnp.min(jnp.where(l2 == v2, ids, E), axis=1, keepdims=True)
    s = jnp.exp(v2 - v1)
    w_ref[...] = jnp.concatenate([1.0 / (1.0 + s), s / (1.0 + s)], axis=1)
    e_ref[...] = jnp.concatenate([i1, i2], axis=1)


def _gate(x, Wg, interpret=False):
    return pl.pallas_call(
        _gate_body,
        out_shape=(jax.ShapeDtypeStruct((N, TOPK), jnp.float32),
                   jax.ShapeDtypeStruct((N, TOPK), jnp.int32)),
        interpret=interpret,
    )(x, Wg)


def _dispatch_body(tok_ref, x_ref, xs_ref):
    t = pl.program_id(0)

    def body(r, _):
        tk = tok_ref[t * T + r]
        xs_ref[pl.ds(r, 1), :] = x_ref[pl.ds(tk, 1), :]
        return 0

    jax.lax.fori_loop(0, T, body, 0)


def _dispatch(x, row_token, interpret=False):
    grid_spec = pltpu.PrefetchScalarGridSpec(
        num_scalar_prefetch=1,
        grid=(NUM_TILES,),
        in_specs=[pl.BlockSpec((N, D_MODEL), lambda t, tok: (0, 0))],
        out_specs=pl.BlockSpec((T, D_MODEL), lambda t, tok: (t, 0)),
    )
    return pl.pallas_call(
        _dispatch_body,
        grid_spec=grid_spec,
        out_shape=jax.ShapeDtypeStruct((NUM_ROWS, D_MODEL), jnp.float32),
        interpret=interpret,
    )(row_token, x)


def _moe_body(te_ref, tv_ref, xs_ref, w_ref, v_ref, w2_ref, rw_ref, out_ref,
              acc_ref):
    f = pl.program_id(0)
    t = pl.program_id(1)
    rows = pl.ds(t * T, T)

    @pl.when(f == 0)
    def _zero():
        acc_ref[rows, :] = jnp.zeros((T, D_MODEL), jnp.float32)

    @pl.when(tv_ref[t] == 1)
    def _compute():
        a = xs_ref[...].astype(jnp.bfloat16)
        h1 = jax.lax.dot_general(
            a, w_ref[0].astype(jnp.bfloat16), (((1,), (1,)), ((), ())),
            preferred_element_type=jnp.float32)
        h2 = jax.lax.dot_general(
            a, v_ref[0].astype(jnp.bfloat16), (((1,), (1,)), ((), ())),
            preferred_element_type=jnp.float32)
        h = ((h1 * jax.nn.sigmoid(h1)) * h2).astype(jnp.bfloat16)
        acc_ref[rows, :] += jax.lax.dot_general(
            h, w2_ref[0].astype(jnp.bfloat16), (((1,), (1,)), ((), ())),
            preferred_element_type=jnp.float32)

    @pl.when(f == NUM_F - 1)
    def _scale():
        out_ref[...] = acc_ref[rows, :] * rw_ref[0, 0, :][:, None]


def _moe_gemm(xs, W, V, W2, row_weight, tile_expert, tile_valid,
              interpret=False):
    last = NUM_F - 1
    grid_spec = pltpu.PrefetchScalarGridSpec(
        num_scalar_prefetch=2,
        grid=(NUM_F, NUM_TILES),
        in_specs=[
            pl.BlockSpec((T, D_MODEL), lambda f, t, te, tv: (t, 0)),
            pl.BlockSpec((1, F_BLK, D_MODEL), lambda f, t, te, tv: (te[t], f, 0)),
            pl.BlockSpec((1, F_BLK, D_MODEL), lambda f, t, te, tv: (te[t], f, 0)),
            pl.BlockSpec((1, D_MODEL, F_BLK), lambda f, t, te, tv: (te[t], 0, f)),
            pl.BlockSpec((1, 1, T), lambda f, t, te, tv: (t, 0, 0)),
        ],
        out_specs=pl.BlockSpec(
            (T, D_MODEL),
            lambda f, t, te, tv: (jnp.where(f == last, t, NUM_TILES), 0)),
        scratch_shapes=[pltpu.VMEM((NUM_ROWS, D_MODEL), jnp.float32)],
    )
    outs = pl.pallas_call(
        _moe_body,
        grid_spec=grid_spec,
        out_shape=jax.ShapeDtypeStruct(((NUM_TILES + 1) * T, D_MODEL),
                                       jnp.float32),
        interpret=interpret,
    )(tile_expert, tile_valid, xs, W, V, W2,
      row_weight.reshape(NUM_TILES, 1, T))
    return outs[:NUM_ROWS]


def _combine_body(tok_ref, outs_ref, y_ref):
    t = pl.program_id(0)

    @pl.when(t == 0)
    def _zero():
        y_ref[...] = jnp.zeros_like(y_ref)

    def body(r, _):
        tk = tok_ref[t * T + r]
        y_ref[pl.ds(tk, 1), :] += outs_ref[pl.ds(r, 1), :]
        return 0

    jax.lax.fori_loop(0, T, body, 0)


def _combine(outs, row_token, interpret=False):
    grid_spec = pltpu.PrefetchScalarGridSpec(
        num_scalar_prefetch=1,
        grid=(NUM_TILES,),
        in_specs=[pl.BlockSpec((T, D_MODEL), lambda t, tok: (t, 0))],
        out_specs=pl.BlockSpec((N, D_MODEL), lambda t, tok: (0, 0)),
    )
    return pl.pallas_call(
        _combine_body,
        grid_spec=grid_spec,
        out_shape=jax.ShapeDtypeStruct((N, D_MODEL), jnp.float32),
        interpret=interpret,
    )(row_token, outs)


def _routing_metadata(weights, experts):
    e_flat = experts.reshape(-1).astype(jnp.int32)
    w_flat = weights.reshape(-1)
    tok = jnp.arange(NTOT, dtype=jnp.int32) // TOPK
    order = jnp.argsort(e_flat, stable=True)
    e_sorted = e_flat[order]
    counts = jnp.bincount(e_flat, length=E).astype(jnp.int32)
    tiles_per_e = (counts + T - 1) // T
    tile_start = jnp.concatenate(
        [jnp.zeros(1, jnp.int32), jnp.cumsum(tiles_per_e)])
    count_start = jnp.concatenate(
        [jnp.zeros(1, jnp.int32), jnp.cumsum(counts)])
    k = jnp.arange(NTOT, dtype=jnp.int32)
    row = tile_start[e_sorted] * T + (k - count_start[e_sorted])
    row_token = jnp.zeros(NUM_ROWS, jnp.int32).at[row].set(tok[order])
    row_weight = jnp.zeros(NUM_ROWS, jnp.float32).at[row].set(w_flat[order])
    total_tiles = tile_start[E]
    g = jnp.minimum(jnp.arange(NUM_TILES, dtype=jnp.int32), total_tiles - 1)
    tile_expert = (jnp.searchsorted(tile_start, g, side="right") - 1
                   ).astype(jnp.int32)
    tile_valid = (jnp.arange(NUM_TILES, dtype=jnp.int32)
                  < total_tiles).astype(jnp.int32)
    return row_token, row_weight, tile_expert, tile_valid


@functools.partial(jax.jit, static_argnames=("interpret",))
def kernel(x, Wg, W, V, W2, interpret=False):
    weights, experts = _gate(x, Wg, interpret)
    row_token, row_weight, tile_expert, tile_valid = _routing_metadata(
        weights, experts)
    xs = _dispatch(x, row_token, interpret)
    outs = _moe_gemm(xs, W, V, W2, row_weight, tile_expert, tile_valid,
                     interpret)
    return _combine(outs, row_token, interpret)


# speedup vs baseline: 1.2086x; 1.0489x over previous
import functools

import jax
import jax.numpy as jnp
from jax.experimental import pallas as pl
from jax.experimental.pallas import tpu as pltpu

D_MODEL = 1024
D_FF = 4096
E = 8
TOPK = 2
N = 2048

T = 128
NTOT = N * TOPK
NUM_TILES = NTOT // T + E
NUM_ROWS = NUM_TILES * T
F_BLK = 512
NUM_F = D_FF // F_BLK


def _gate_body(x_ref, wg_ref, w_ref, e_ref):
    logits = jax.lax.dot_general(
        x_ref[...], wg_ref[...], (((1,), (1,)), ((), ())),
        preferred_element_type=jnp.float32)
    ids = jax.lax.broadcasted_iota(jnp.int32, logits.shape, 1)
    v1 = jnp.max(logits, axis=1, keepdims=True)
    i1 = jnp.min(jnp.where(logits == v1, ids, E), axis=1, keepdims=True)
    l2 = jnp.where(ids == i1, -jnp.inf, logits)
    v2 = jnp.max(l2, axis=1, keepdims=True)
    i2 = jnp.min(jnp.where(l2 == v2, ids, E), axis=1, keepdims=True)
    s = jnp.exp(v2 - v1)
    w_ref[...] = jnp.concatenate([1.0 / (1.0 + s), s / (1.0 + s)], axis=1)
    e_ref[...] = jnp.concatenate([i1, i2], axis=1)


def _gate(x, Wg, interpret=False):
    return pl.pallas_call(
        _gate_body,
        out_shape=(jax.ShapeDtypeStruct((N, TOPK), jnp.float32),
                   jax.ShapeDtypeStruct((N, TOPK), jnp.int32)),
        interpret=interpret,
    )(x, Wg)


def _dispatch_body(p0_ref, p1_ref, x_ref, xs_ref):
    t = pl.program_id(0)
    rowid = t * T + jax.lax.broadcasted_iota(jnp.int32, (1, T), 1)
    st = jnp.logical_or(p0_ref[...] == rowid, p1_ref[...] == rowid)
    stb = st.astype(jnp.bfloat16)
    xs_ref[...] = jax.lax.dot_general(
        stb, x_ref[...], (((0,), (0,)), ((), ())),
        preferred_element_type=jnp.float32).astype(jnp.bfloat16)


def _dispatch(x_bf, pos0, pos1, interpret=False):
    return pl.pallas_call(
        _dispatch_body,
        grid=(NUM_TILES,),
        in_specs=[
            pl.BlockSpec((N, 1), lambda t: (0, 0)),
            pl.BlockSpec((N, 1), lambda t: (0, 0)),
            pl.BlockSpec((N, D_MODEL), lambda t: (0, 0)),
        ],
        out_specs=pl.BlockSpec((T, D_MODEL), lambda t: (t, 0)),
        out_shape=jax.ShapeDtypeStruct((NUM_ROWS, D_MODEL), jnp.bfloat16),
        interpret=interpret,
    )(pos0, pos1, x_bf)


def _moe_body(te_ref, tv_ref, xs_ref, w_ref, v_ref, w2_ref, rw_ref, out_ref,
              acc_ref):
    f = pl.program_id(0)
    t = pl.program_id(1)
    rows = pl.ds(t * T, T)

    @pl.when(f == 0)
    def _zero():
        acc_ref[rows, :] = jnp.zeros((T, D_MODEL), jnp.float32)

    @pl.when(tv_ref[t] == 1)
    def _compute():
        a = xs_ref[...]
        h1 = jax.lax.dot_general(
            a, w_ref[0].astype(jnp.bfloat16), (((1,), (1,)), ((), ())),
            preferred_element_type=jnp.float32)
        h2 = jax.lax.dot_general(
            a, v_ref[0].astype(jnp.bfloat16), (((1,), (1,)), ((), ())),
            preferred_element_type=jnp.float32)
        h = ((h1 * jax.nn.sigmoid(h1)) * h2).astype(jnp.bfloat16)
        acc_ref[rows, :] += jax.lax.dot_general(
            h, w2_ref[0].astype(jnp.bfloat16), (((1,), (1,)), ((), ())),
            preferred_element_type=jnp.float32)

    @pl.when(f == NUM_F - 1)
    def _scale():
        out_ref[...] = (acc_ref[rows, :]
                        * rw_ref[0, 0, :][:, None]).astype(jnp.bfloat16)


def _moe_gemm(xs, W, V, W2, row_weight, tile_expert, tile_valid,
              interpret=False):
    last = NUM_F - 1
    grid_spec = pltpu.PrefetchScalarGridSpec(
        num_scalar_prefetch=2,
        grid=(NUM_F, NUM_TILES),
        in_specs=[
            pl.BlockSpec((T, D_MODEL), lambda f, t, te, tv: (t, 0)),
            pl.BlockSpec((1, F_BLK, D_MODEL), lambda f, t, te, tv: (te[t], f, 0)),
            pl.BlockSpec((1, F_BLK, D_MODEL), lambda f, t, te, tv: (te[t], f, 0)),
            pl.BlockSpec((1, D_MODEL, F_BLK), lambda f, t, te, tv: (te[t], 0, f)),
            pl.BlockSpec((1, 1, T), lambda f, t, te, tv: (t, 0, 0)),
        ],
        out_specs=pl.BlockSpec(
            (T, D_MODEL),
            lambda f, t, te, tv: (jnp.where(f == last, t, NUM_TILES), 0)),
        scratch_shapes=[pltpu.VMEM((NUM_ROWS, D_MODEL), jnp.float32)],
    )
    outs = pl.pallas_call(
        _moe_body,
        grid_spec=grid_spec,
        out_shape=jax.ShapeDtypeStruct(((NUM_TILES + 1) * T, D_MODEL),
                                       jnp.bfloat16),
        interpret=interpret,
    )(tile_expert, tile_valid, xs, W, V, W2,
      row_weight.reshape(NUM_TILES, 1, T))
    return outs[:NUM_ROWS]


NB = 128
def _combine_body(p0_ref, p1_ref, outs_ref, y_ref):
    rowid = jax.lax.broadcasted_iota(jnp.int32, (1, NUM_ROWS), 1)
    st = jnp.logical_or(p0_ref[...] == rowid, p1_ref[...] == rowid)
    stb = st.astype(jnp.bfloat16)
    y_ref[...] = jax.lax.dot_general(
        stb, outs_ref[...], (((1,), (0,)), ((), ())),
        preferred_element_type=jnp.float32)


def _combine(outs, pos0, pos1, interpret=False):
    return pl.pallas_call(
        _combine_body,
        grid=(N // NB,),
        in_specs=[
            pl.BlockSpec((NB, 1), lambda b: (b, 0)),
            pl.BlockSpec((NB, 1), lambda b: (b, 0)),
            pl.BlockSpec((NUM_ROWS, D_MODEL), lambda b: (0, 0)),
        ],
        out_specs=pl.BlockSpec((NB, D_MODEL), lambda b: (b, 0)),
        out_shape=jax.ShapeDtypeStruct((N, D_MODEL), jnp.float32),
        interpret=interpret,
    )(pos0, pos1, outs)


def _routing_metadata(weights, experts):
    e_flat = experts.T.reshape(-1).astype(jnp.int32)
    w_flat = weights.T.reshape(-1)
    order = jnp.argsort(e_flat, stable=True)
    e_sorted = e_flat[order]
    counts = jnp.bincount(e_flat, length=E).astype(jnp.int32)
    tiles_per_e = (counts + T - 1) // T
    tile_start = jnp.concatenate(
        [jnp.zeros(1, jnp.int32), jnp.cumsum(tiles_per_e)])
    count_start = jnp.concatenate(
        [jnp.zeros(1, jnp.int32), jnp.cumsum(counts)])
    k = jnp.arange(NTOT, dtype=jnp.int32)
    row = tile_start[e_sorted] * T + (k - count_start[e_sorted])
    pos = jnp.zeros(NTOT, jnp.int32).at[order].set(row)
    row_weight = jnp.zeros(NUM_ROWS, jnp.float32).at[row].set(w_flat[order])
    total_tiles = tile_start[E]
    g = jnp.minimum(jnp.arange(NUM_TILES, dtype=jnp.int32), total_tiles - 1)
    tile_expert = (jnp.searchsorted(tile_start, g, side="right") - 1
                   ).astype(jnp.int32)
    tile_valid = (jnp.arange(NUM_TILES, dtype=jnp.int32)
                  < total_tiles).astype(jnp.int32)
    pos0 = pos[:N].reshape(N, 1)
    pos1 = pos[N:].reshape(N, 1)
    return pos0, pos1, row_weight, tile_expert, tile_valid


@functools.partial(jax.jit, static_argnames=("interpret",))
def kernel(x, Wg, W, V, W2, interpret=False):
    weights, experts = _gate(x, Wg, interpret)
    pos0, pos1, row_weight, tile_expert, tile_valid = _routing_metadata(
        weights, experts)
    xs = _dispatch(x.astype(jnp.bfloat16), pos0, pos1, interpret)
    outs = _moe_gemm(xs, W, V, W2, row_weight, tile_expert, tile_valid,
                     interpret)
    return _combine(outs, pos0, pos1, interpret)


# speedup vs baseline: 1.8252x; 1.5103x over previous
import functools

import jax
import jax.numpy as jnp
from jax.experimental import pallas as pl
from jax.experimental.pallas import tpu as pltpu

D_MODEL = 1024
D_FF = 4096
E = 8
TOPK = 2
N = 2048

T = 512
NTOT = N * TOPK
NUM_TILES = NTOT // T + E - 1
NUM_ROWS = NUM_TILES * T
F_BLK = 512
NUM_F = D_FF // F_BLK


def _gate_body(x_ref, wg_ref, w_ref, e_ref):
    logits = jax.lax.dot_general(
        x_ref[...], wg_ref[...], (((1,), (1,)), ((), ())),
        preferred_element_type=jnp.float32)
    ids = jax.lax.broadcasted_iota(jnp.int32, logits.shape, 1)
    v1 = jnp.max(logits, axis=1, keepdims=True)
    i1 = jnp.min(jnp.where(logits == v1, ids, E), axis=1, keepdims=True)
    l2 = jnp.where(ids == i1, -jnp.inf, logits)
    v2 = jnp.max(l2, axis=1, keepdims=True)
    i2 = jnp.min(jnp.where(l2 == v2, ids, E), axis=1, keepdims=True)
    s = jnp.exp(v2 - v1)
    w_ref[...] = jnp.concatenate([1.0 / (1.0 + s), s / (1.0 + s)], axis=1)
    e_ref[...] = jnp.concatenate([i1, i2], axis=1)


def _gate(x, Wg, interpret=False):
    return pl.pallas_call(
        _gate_body,
        out_shape=(jax.ShapeDtypeStruct((N, TOPK), jnp.float32),
                   jax.ShapeDtypeStruct((N, TOPK), jnp.int32)),
        interpret=interpret,
    )(x, Wg)


def _dispatch_body(p0_ref, p1_ref, x_ref, xs_ref):
    t = pl.program_id(0)
    rowid = t * T + jax.lax.broadcasted_iota(jnp.int32, (1, T), 1)
    st = jnp.logical_or(p0_ref[...] == rowid, p1_ref[...] == rowid)
    stb = st.astype(jnp.bfloat16)
    xs_ref[...] = jax.lax.dot_general(
        stb, x_ref[...], (((0,), (0,)), ((), ())),
        preferred_element_type=jnp.float32).astype(jnp.bfloat16)


def _dispatch(x_bf, pos0, pos1, interpret=False):
    return pl.pallas_call(
        _dispatch_body,
        grid=(NUM_TILES,),
        in_specs=[
            pl.BlockSpec((N, 1), lambda t: (0, 0)),
            pl.BlockSpec((N, 1), lambda t: (0, 0)),
            pl.BlockSpec((N, D_MODEL), lambda t: (0, 0)),
        ],
        out_specs=pl.BlockSpec((T, D_MODEL), lambda t: (t, 0)),
        out_shape=jax.ShapeDtypeStruct((NUM_ROWS, D_MODEL), jnp.bfloat16),
        interpret=interpret,
    )(pos0, pos1, x_bf)


def _moe_body(te_ref, tv_ref, xs_ref, w_ref, v_ref, w2_ref, rw_ref, out_ref,
              acc_ref):
    t = pl.program_id(0)
    f = pl.program_id(1)

    @pl.when(tv_ref[t] == 1)
    def _compute():
        a = xs_ref[...]
        h1 = jax.lax.dot_general(
            a, w_ref[0].astype(jnp.bfloat16), (((1,), (1,)), ((), ())),
            preferred_element_type=jnp.float32)
        h2 = jax.lax.dot_general(
            a, v_ref[0].astype(jnp.bfloat16), (((1,), (1,)), ((), ())),
            preferred_element_type=jnp.float32)
        h = ((h1 * jax.nn.sigmoid(h1)) * h2).astype(jnp.bfloat16)
        contrib = jax.lax.dot_general(
            h, w2_ref[0].astype(jnp.bfloat16), (((1,), (1,)), ((), ())),
            preferred_element_type=jnp.float32)
        if_first = f == 0

        @pl.when(if_first)
        def _init():
            acc_ref[...] = contrib

        @pl.when(jnp.logical_not(if_first))
        def _add():
            acc_ref[...] += contrib

    @pl.when(f == NUM_F - 1)
    def _scale():
        res = jnp.where(tv_ref[t] == 1, acc_ref[...], 0.0)
        out_ref[...] = (res * rw_ref[0, 0, :][:, None]).astype(jnp.bfloat16)


def _moe_gemm(xs, W, V, W2, row_weight, tile_expert, tile_valid,
              interpret=False):
    grid_spec = pltpu.PrefetchScalarGridSpec(
        num_scalar_prefetch=2,
        grid=(NUM_TILES, NUM_F),
        in_specs=[
            pl.BlockSpec((T, D_MODEL), lambda t, f, te, tv: (t, 0)),
            pl.BlockSpec((1, F_BLK, D_MODEL), lambda t, f, te, tv: (te[t], f, 0)),
            pl.BlockSpec((1, F_BLK, D_MODEL), lambda t, f, te, tv: (te[t], f, 0)),
            pl.BlockSpec((1, D_MODEL, F_BLK), lambda t, f, te, tv: (te[t], 0, f)),
            pl.BlockSpec((1, 1, T), lambda t, f, te, tv: (t, 0, 0)),
        ],
        out_specs=pl.BlockSpec((T, D_MODEL), lambda t, f, te, tv: (t, 0)),
        scratch_shapes=[pltpu.VMEM((T, D_MODEL), jnp.float32)],
    )
    return pl.pallas_call(
        _moe_body,
        grid_spec=grid_spec,
        out_shape=jax.ShapeDtypeStruct((NUM_ROWS, D_MODEL), jnp.bfloat16),
        interpret=interpret,
    )(tile_expert, tile_valid, xs, W, V, W2,
      row_weight.reshape(NUM_TILES, 1, T))


NB = 128
def _combine_body(p0_ref, p1_ref, outs_ref, y_ref):
    rowid = jax.lax.broadcasted_iota(jnp.int32, (1, NUM_ROWS), 1)
    st = jnp.logical_or(p0_ref[...] == rowid, p1_ref[...] == rowid)
    stb = st.astype(jnp.bfloat16)
    y_ref[...] = jax.lax.dot_general(
        stb, outs_ref[...], (((1,), (0,)), ((), ())),
        preferred_element_type=jnp.float32)


def _combine(outs, pos0, pos1, interpret=False):
    return pl.pallas_call(
        _combine_body,
        grid=(N // NB,),
        in_specs=[
            pl.BlockSpec((NB, 1), lambda b: (b, 0)),
            pl.BlockSpec((NB, 1), lambda b: (b, 0)),
            pl.BlockSpec((NUM_ROWS, D_MODEL), lambda b: (0, 0)),
        ],
        out_specs=pl.BlockSpec((NB, D_MODEL), lambda b: (b, 0)),
        out_shape=jax.ShapeDtypeStruct((N, D_MODEL), jnp.float32),
        interpret=interpret,
    )(pos0, pos1, outs)


def _routing_metadata(weights, experts):
    e_flat = experts.T.reshape(-1).astype(jnp.int32)
    w_flat = weights.T.reshape(-1)
    order = jnp.argsort(e_flat, stable=True)
    e_sorted = e_flat[order]
    counts = jnp.bincount(e_flat, length=E).astype(jnp.int32)
    tiles_per_e = (counts + T - 1) // T
    tile_start = jnp.concatenate(
        [jnp.zeros(1, jnp.int32), jnp.cumsum(tiles_per_e)])
    count_start = jnp.concatenate(
        [jnp.zeros(1, jnp.int32), jnp.cumsum(counts)])
    k = jnp.arange(NTOT, dtype=jnp.int32)
    row = tile_start[e_sorted] * T + (k - count_start[e_sorted])
    pos = jnp.zeros(NTOT, jnp.int32).at[order].set(row)
    row_weight = jnp.zeros(NUM_ROWS, jnp.float32).at[row].set(w_flat[order])
    total_tiles = tile_start[E]
    g = jnp.minimum(jnp.arange(NUM_TILES, dtype=jnp.int32), total_tiles - 1)
    tile_expert = (jnp.searchsorted(tile_start, g, side="right") - 1
                   ).astype(jnp.int32)
    tile_valid = (jnp.arange(NUM_TILES, dtype=jnp.int32)
                  < total_tiles).astype(jnp.int32)
    pos0 = pos[:N].reshape(N, 1)
    pos1 = pos[N:].reshape(N, 1)
    return pos0, pos1, row_weight, tile_expert, tile_valid


@functools.partial(jax.jit, static_argnames=("interpret",))
def kernel(x, Wg, W, V, W2, interpret=False):
    weights, experts = _gate(x, Wg, interpret)
    pos0, pos1, row_weight, tile_expert, tile_valid = _routing_metadata(
        weights, experts)
    xs = _dispatch(x.astype(jnp.bfloat16), pos0, pos1, interpret)
    outs = _moe_gemm(xs, W, V, W2, row_weight, tile_expert, tile_valid,
                     interpret)
    return _combine(outs, pos0, pos1, interpret)
